# initial kernel scaffold (unmeasured)
import jax
import jax.numpy as jnp
from jax import lax
from jax.experimental import pallas as pl
from jax.experimental.pallas import tpu as pltpu

N_DEV = 8
M_PER = 512
K = 4096
N = 8192
NB = 1024
N_STEPS = N // NB


def kernel(x, w_mat):
    assert x.shape == (K, K // N_DEV), x.shape
    assert w_mat.shape == (K, N), w_mat.shape

    def body(x_ref, w_ref, out_ref, x_rows, gather, acc,
             send_sems, recv_sems, a_send_sems, a_recv_sems):
        j = pl.program_id(0)
        my = lax.axis_index("i")

        def x_send_rdma(d):
            tgt = lax.rem(my + d, N_DEV)
            return pltpu.make_async_remote_copy(
                src_ref=x_ref.at[pl.ds(tgt * M_PER, M_PER), :],
                dst_ref=x_rows.at[:, pl.ds(my * (K // N_DEV), K // N_DEV)],
                send_sem=send_sems.at[d],
                recv_sem=recv_sems.at[d],
                device_id=(tgt,),
                device_id_type=pl.DeviceIdType.MESH,
            )

        def amax_send_rdma(d):
            tgt = lax.rem(my + d, N_DEV)
            return pltpu.make_async_remote_copy(
                src_ref=gather.at[pl.ds(0, 1), :],
                dst_ref=gather.at[pl.ds(d, 1), :],
                send_sem=a_send_sems.at[d],
                recv_sem=a_recv_sems.at[d],
                device_id=(tgt,),
                device_id_type=pl.DeviceIdType.MESH,
            )

        @pl.when(j == 0)
        def _comm():
            barrier = pltpu.get_barrier_semaphore()
            for p in range(N_DEV):
                pl.semaphore_signal(
                    barrier, inc=1, device_id=(p,),
                    device_id_type=pl.DeviceIdType.MESH,
                )
            pl.semaphore_wait(barrier, N_DEV)

            for d in range(1, N_DEV):
                x_send_rdma(d).start()

            x_rows[:, pl.ds(my * (K // N_DEV), K // N_DEV)] = (
                x_ref[pl.ds(my * M_PER, M_PER), :]
            )

            for d in range(1, N_DEV):
                src = lax.rem(my - d + N_DEV, N_DEV)
                rdma = pltpu.make_async_remote_copy(
                    src_ref=x_ref.at[pl.ds(src * M_PER, M_PER), :],
                    dst_ref=x_rows.at[:, pl.ds(src * (K // N_DEV), K // N_DEV)],
                    send_sem=send_sems.at[d],
                    recv_sem=recv_sems.at[d],
                    device_id=(src,),
                    device_id_type=pl.DeviceIdType.MESH,
                )
                rdma.wait_recv()

            acc[0] = 0.0

        yblk = jnp.maximum(
            jnp.dot(x_rows[:, :], w_ref[:, :],
                    preferred_element_type=jnp.float32),
            0.0,
        )
        out_ref[:, pl.ds(j * NB, NB)] = yblk
        acc[0] = jnp.maximum(acc[0], jnp.max(yblk))

        @pl.when(j == N_STEPS - 1)
        def _epilogue():
            for d in range(1, N_DEV):
                x_send_rdma(d).wait_send()

            gather[pl.ds(0, 1), :] = jnp.full((1, 128), acc[0], jnp.float32)
            for d in range(1, N_DEV):
                amax_send_rdma(d).start()
            for d in range(1, N_DEV):
                amax_send_rdma(d).wait()

            g = jnp.maximum(jnp.max(gather[:, :]), 1e-30)
            scale = g / 448.0
            q = (out_ref[:, :] * (448.0 / g)).astype(jnp.float8_e4m3fn)
            out_ref[:, :] = q.astype(jnp.float32) * scale

    return pl.pallas_call(
        body,
        grid=(N_STEPS,),
        out_shape=jax.ShapeDtypeStruct((M_PER, N), jnp.float32),
        in_specs=[
            pl.BlockSpec((K, K // N_DEV), lambda j: (0, 0),
                         memory_space=pltpu.VMEM),
            pl.BlockSpec((K, NB), lambda j: (0, j),
                         memory_space=pltpu.VMEM),
        ],
        out_specs=pl.BlockSpec((M_PER, N), lambda j: (0, 0),
                               memory_space=pltpu.VMEM),
        scratch_shapes=[
            pltpu.VMEM((M_PER, K), jnp.float32),
            pltpu.VMEM((N_DEV, 128), jnp.float32),
            pltpu.SMEM((1,), jnp.float32),
            pltpu.SemaphoreType.DMA((N_DEV,)),
            pltpu.SemaphoreType.DMA((N_DEV,)),
            pltpu.SemaphoreType.DMA((N_DEV,)),
            pltpu.SemaphoreType.DMA((N_DEV,)),
        ],
        compiler_params=pltpu.CompilerParams(collective_id=0),
    )(x, w_mat)


# baseline (device time: 147917 ns/iter reference)
import jax
import jax.numpy as jnp
from jax import lax
from jax.experimental import pallas as pl
from jax.experimental.pallas import tpu as pltpu

N_DEV = 8
M_PER = 512
K = 4096
N = 8192
NB = 512
N_STEPS = N // NB


def kernel(x, w_mat):
    assert x.shape == (K, K // N_DEV), x.shape
    assert w_mat.shape == (K, N), w_mat.shape

    def body(x_ref, w_ref, out_ref, x_rows, gather, acc,
             send_sems, recv_sems, a_send_sems, a_recv_sems):
        j = pl.program_id(0)
        my = lax.axis_index("i")

        def x_send_rdma(d):
            tgt = lax.rem(my + d, N_DEV)
            return pltpu.make_async_remote_copy(
                src_ref=x_ref.at[pl.ds(tgt * M_PER, M_PER), :],
                dst_ref=x_rows.at[:, pl.ds(my * (K // N_DEV), K // N_DEV)],
                send_sem=send_sems.at[d],
                recv_sem=recv_sems.at[d],
                device_id=(tgt,),
                device_id_type=pl.DeviceIdType.MESH,
            )

        def amax_send_rdma(d):
            tgt = lax.rem(my + d, N_DEV)
            return pltpu.make_async_remote_copy(
                src_ref=gather.at[pl.ds(0, 1), :],
                dst_ref=gather.at[pl.ds(d, 1), :],
                send_sem=a_send_sems.at[d],
                recv_sem=a_recv_sems.at[d],
                device_id=(tgt,),
                device_id_type=pl.DeviceIdType.MESH,
            )

        @pl.when(j == 0)
        def _comm():
            barrier = pltpu.get_barrier_semaphore()
            for p in range(N_DEV):
                @pl.when(p != my)
                def _signal():
                    pl.semaphore_signal(
                        barrier, inc=1, device_id=(p,),
                        device_id_type=pl.DeviceIdType.MESH,
                    )
            pl.semaphore_wait(barrier, N_DEV - 1)

            for d in range(1, N_DEV):
                x_send_rdma(d).start()

            x_rows[:, pl.ds(my * (K // N_DEV), K // N_DEV)] = (
                x_ref[pl.ds(my * M_PER, M_PER), :]
            )

            for d in range(1, N_DEV):
                src = lax.rem(my - d + N_DEV, N_DEV)
                rdma = pltpu.make_async_remote_copy(
                    src_ref=x_ref.at[pl.ds(src * M_PER, M_PER), :],
                    dst_ref=x_rows.at[:, pl.ds(src * (K // N_DEV), K // N_DEV)],
                    send_sem=send_sems.at[d],
                    recv_sem=recv_sems.at[d],
                    device_id=(src,),
                    device_id_type=pl.DeviceIdType.MESH,
                )
                rdma.wait_recv()

            acc[0] = 0.0

        yblk = jnp.maximum(
            jnp.dot(x_rows[:, :], w_ref[:, :],
                    preferred_element_type=jnp.float32),
            0.0,
        )
        out_ref[:, pl.ds(j * NB, NB)] = yblk
        acc[0] = jnp.maximum(acc[0], jnp.max(yblk))

        @pl.when(j == N_STEPS - 1)
        def _epilogue():
            for d in range(1, N_DEV):
                x_send_rdma(d).wait_send()

            gather[pl.ds(0, 1), :] = jnp.full((1, 128), acc[0], jnp.float32)
            for d in range(1, N_DEV):
                amax_send_rdma(d).start()
            for d in range(1, N_DEV):
                amax_send_rdma(d).wait()

            g = jnp.maximum(jnp.max(gather[:, :]), 1e-30)
            scale = g / 448.0
            for c in range(0, N, 2048):
                blk = out_ref[:, pl.ds(c, 2048)]
                q = (blk * (448.0 / g)).astype(jnp.float8_e4m3fn)
                out_ref[:, pl.ds(c, 2048)] = q.astype(jnp.float32) * scale

    return pl.pallas_call(
        body,
        grid=(N_STEPS,),
        out_shape=jax.ShapeDtypeStruct((M_PER, N), jnp.float32),
        in_specs=[
            pl.BlockSpec((K, K // N_DEV), lambda j: (0, 0),
                         memory_space=pltpu.VMEM),
            pl.BlockSpec((K, NB), lambda j: (0, j),
                         memory_space=pltpu.VMEM),
        ],
        out_specs=pl.BlockSpec((M_PER, N), lambda j: (0, 0),
                               memory_space=pltpu.VMEM),
        scratch_shapes=[
            pltpu.VMEM((M_PER, K), jnp.float32),
            pltpu.VMEM((N_DEV, 128), jnp.float32),
            pltpu.SMEM((1,), jnp.float32),
            pltpu.SemaphoreType.DMA((N_DEV,)),
            pltpu.SemaphoreType.DMA((N_DEV,)),
            pltpu.SemaphoreType.DMA((N_DEV,)),
            pltpu.SemaphoreType.DMA((N_DEV,)),
        ],
        compiler_params=pltpu.CompilerParams(
            collective_id=0,
            vmem_limit_bytes=63 * 1024 * 1024,
        ),
    )(x, w_mat)


# device time: 107992 ns/iter; 1.3697x vs baseline; 1.3697x over previous
import os

import jax
import jax.numpy as jnp
from jax import lax
from jax.experimental import pallas as pl
from jax.experimental.pallas import tpu as pltpu

_SKIP_A2A = bool(int(os.environ.get("SKIP_A2A", "0")))
_SKIP_QUANT = bool(int(os.environ.get("SKIP_QUANT", "0")))

N_DEV = 8
M_PER = 512
K = 4096
KC = K // N_DEV
N = 8192
NB = 1024


def kernel(x, w_mat):
    assert x.shape == (K, KC), x.shape
    assert w_mat.shape == (K, N), w_mat.shape

    def body(x_hbm, w_hbm, out_ref, x_rows, w_bufs, gather, acc,
             send_sems, recv_sems, a_send_sems, a_recv_sems, copy_sems):
        my = lax.axis_index("i")

        def x_send_rdma(d):
            tgt = lax.rem(my + d, N_DEV)
            return pltpu.make_async_remote_copy(
                src_ref=x_hbm.at[pl.ds(tgt * M_PER, M_PER), :],
                dst_ref=x_rows.at[:, pl.ds(my * KC, KC)],
                send_sem=send_sems.at[d],
                recv_sem=recv_sems.at[d],
                device_id=(tgt,),
                device_id_type=pl.DeviceIdType.MESH,
            )

        def x_recv_rdma(d):
            src = lax.rem(my - d + N_DEV, N_DEV)
            return pltpu.make_async_remote_copy(
                src_ref=x_hbm.at[pl.ds(src * M_PER, M_PER), :],
                dst_ref=x_rows.at[:, pl.ds(src * KC, KC)],
                send_sem=send_sems.at[d],
                recv_sem=recv_sems.at[d],
                device_id=(src,),
                device_id_type=pl.DeviceIdType.MESH,
            )

        def w_copy(d, slot):
            src = lax.rem(my - d + N_DEV, N_DEV)
            return pltpu.make_async_copy(
                w_hbm.at[pl.ds(src * KC, KC), :],
                w_bufs.at[slot],
                copy_sems.at[slot],
            )

        def amax_send_rdma(d):
            tgt = lax.rem(my + d, N_DEV)
            return pltpu.make_async_remote_copy(
                src_ref=gather.at[pl.ds(0, 1), :],
                dst_ref=gather.at[pl.ds(d, 1), :],
                send_sem=a_send_sems.at[d],
                recv_sem=a_recv_sems.at[d],
                device_id=(tgt,),
                device_id_type=pl.DeviceIdType.MESH,
            )

        barrier = pltpu.get_barrier_semaphore()
        for p in range(N_DEV):
            @pl.when(p != my)
            def _signal():
                pl.semaphore_signal(
                    barrier, inc=1, device_id=(p,),
                    device_id_type=pl.DeviceIdType.MESH,
                )
        pl.semaphore_wait(barrier, N_DEV - 1)

        if not _SKIP_A2A:
            for d in range(1, N_DEV):
                x_send_rdma(d).start()

        local_cp = pltpu.make_async_copy(
            x_hbm.at[pl.ds(my * M_PER, M_PER), :],
            x_rows.at[:, pl.ds(my * KC, KC)],
            copy_sems.at[2],
        )
        local_cp.start()
        w_copy(0, 0).start()

        mloc = jnp.float32(0.0)
        for d in range(N_DEV):
            src = lax.rem(my - d + N_DEV, N_DEV)
            slot = d % 2
            if d + 1 < N_DEV:
                w_copy(d + 1, (d + 1) % 2).start()
            w_copy(d, slot).wait()
            if d == 0:
                local_cp.wait()
            elif not _SKIP_A2A:
                x_recv_rdma(d).wait_recv()

            x_chunk = x_rows[:, pl.ds(src * KC, KC)]
            for c in range(0, N, NB):
                part = jnp.dot(
                    x_chunk, w_bufs[slot, :, pl.ds(c, NB)],
                    preferred_element_type=jnp.float32,
                )
                if d == 0:
                    out_ref[:, pl.ds(c, NB)] = part
                elif d < N_DEV - 1:
                    out_ref[:, pl.ds(c, NB)] = out_ref[:, pl.ds(c, NB)] + part
                else:
                    v = jnp.maximum(out_ref[:, pl.ds(c, NB)] + part, 0.0)
                    out_ref[:, pl.ds(c, NB)] = v
                    mloc = jnp.maximum(mloc, jnp.max(v))

        if not _SKIP_A2A:
            for d in range(1, N_DEV):
                x_send_rdma(d).wait_send()

        gather[pl.ds(0, 1), :] = jnp.full((1, 128), mloc, jnp.float32)
        acc[0] = mloc
        for d in range(1, N_DEV):
            amax_send_rdma(d).start()
        for d in range(1, N_DEV):
            amax_send_rdma(d).wait()

        g = jnp.maximum(jnp.max(gather[:, :]), 1e-30)
        scale = g / 448.0
        if not _SKIP_QUANT:
            for c in range(0, N, 2048):
                blk = out_ref[:, pl.ds(c, 2048)]
                q = (blk * (448.0 / g)).astype(jnp.float8_e4m3fn)
                out_ref[:, pl.ds(c, 2048)] = q.astype(jnp.float32) * scale

    return pl.pallas_call(
        body,
        out_shape=jax.ShapeDtypeStruct((M_PER, N), jnp.float32),
        in_specs=[
            pl.BlockSpec(memory_space=pl.ANY),
            pl.BlockSpec(memory_space=pl.ANY),
        ],
        out_specs=pl.BlockSpec(memory_space=pltpu.VMEM),
        scratch_shapes=[
            pltpu.VMEM((M_PER, K), jnp.float32),
            pltpu.VMEM((2, KC, N), jnp.float32),
            pltpu.VMEM((N_DEV, 128), jnp.float32),
            pltpu.SMEM((1,), jnp.float32),
            pltpu.SemaphoreType.DMA((N_DEV,)),
            pltpu.SemaphoreType.DMA((N_DEV,)),
            pltpu.SemaphoreType.DMA((N_DEV,)),
            pltpu.SemaphoreType.DMA((N_DEV,)),
            pltpu.SemaphoreType.DMA((3,)),
        ],
        compiler_params=pltpu.CompilerParams(
            collective_id=0,
            vmem_limit_bytes=63 * 1024 * 1024,
        ),
    )(x, w_mat)


# device time: 84695 ns/iter; 1.7465x vs baseline; 1.2751x over previous
import os

import jax
import jax.numpy as jnp
from jax import lax
from jax.experimental import pallas as pl
from jax.experimental.pallas import tpu as pltpu

_SKIP_A2A = bool(int(os.environ.get("SKIP_A2A", "0")))
_SKIP_QUANT = bool(int(os.environ.get("SKIP_QUANT", "0")))
_SKIP_AMAX = bool(int(os.environ.get("SKIP_AMAX", "0")))
_PROFILE = bool(int(os.environ.get("PROFILE_SCOPES", "0")))

import contextlib


def _scope(name):
    return jax.named_scope(name) if _PROFILE else contextlib.nullcontext()

N_DEV = 8
M_PER = 512
K = 4096
KC = K // N_DEV
N = 8192
NB = 1024


def kernel(x, w_mat):
    assert x.shape == (K, KC), x.shape
    assert w_mat.shape == (K, N), w_mat.shape

    def body(x_hbm, w_hbm, out_ref, x_rows, w_bufs, x_bf16, stage, gather,
             acc, send_sems, recv_sems, a_send_sems, a_recv_sems, copy_sems,
             stage_sems):
        my = lax.axis_index("i")

        def x_send_rdma(d):
            tgt = lax.rem(my + d, N_DEV)
            return pltpu.make_async_remote_copy(
                src_ref=x_bf16.at[pl.ds(tgt * M_PER, M_PER), :],
                dst_ref=x_rows.at[:, pl.ds(my * KC, KC)],
                send_sem=send_sems.at[d],
                recv_sem=recv_sems.at[d],
                device_id=(tgt,),
                device_id_type=pl.DeviceIdType.MESH,
            )

        def x_recv_rdma(d):
            src = lax.rem(my - d + N_DEV, N_DEV)
            return pltpu.make_async_remote_copy(
                src_ref=x_bf16.at[pl.ds(src * M_PER, M_PER), :],
                dst_ref=x_rows.at[:, pl.ds(src * KC, KC)],
                send_sem=send_sems.at[d],
                recv_sem=recv_sems.at[d],
                device_id=(src,),
                device_id_type=pl.DeviceIdType.MESH,
            )

        def stage_copy(d, slot):
            tgt = lax.rem(my + d, N_DEV)
            return pltpu.make_async_copy(
                x_hbm.at[pl.ds(tgt * M_PER, M_PER), :],
                stage.at[slot],
                stage_sems.at[slot],
            )

        N_WSPLIT = 2

        def w_copies(d, slot):
            src = lax.rem(my - d + N_DEV, N_DEV)
            wq = N // N_WSPLIT
            return [
                pltpu.make_async_copy(
                    w_hbm.at[pl.ds(src * KC, KC), pl.ds(h * wq, wq)],
                    w_bufs.at[slot, :, pl.ds(h * wq, wq)],
                    copy_sems.at[slot, h],
                )
                for h in range(N_WSPLIT)
            ]

        def amax_send_rdma(d):
            tgt = lax.rem(my + d, N_DEV)
            return pltpu.make_async_remote_copy(
                src_ref=gather.at[pl.ds(0, 1), :],
                dst_ref=gather.at[pl.ds(d, 1), :],
                send_sem=a_send_sems.at[d],
                recv_sem=a_recv_sems.at[d],
                device_id=(tgt,),
                device_id_type=pl.DeviceIdType.MESH,
            )

        with _scope("entry_barrier"):
            barrier = pltpu.get_barrier_semaphore()
            for p in range(N_DEV):
                @pl.when(p != my)
                def _signal():
                    pl.semaphore_signal(
                        barrier, inc=1, device_id=(p,),
                        device_id_type=pl.DeviceIdType.MESH,
                    )
            pl.semaphore_wait(barrier, N_DEV - 1)

        for cp in w_copies(0, 0):
            cp.start()

        cast_seq = [1, 2, 3, 4, 5, 6, 7, 0]
        stage_copy(cast_seq[0], 0).start()
        stage_copy(cast_seq[1], 1).start()
        for idx, d in enumerate(cast_seq):
            slot = idx % 2
            stage_copy(d, slot).wait()
            blk16 = stage[slot].astype(jnp.bfloat16)
            if d == 0:
                x_rows[:, pl.ds(my * KC, KC)] = blk16
            else:
                tgt = lax.rem(my + d, N_DEV)
                x_bf16[pl.ds(tgt * M_PER, M_PER), :] = blk16
                if not _SKIP_A2A:
                    x_send_rdma(d).start()
            if idx + 2 < len(cast_seq):
                stage_copy(cast_seq[idx + 2], slot).start()

        mloc = jnp.float32(0.0)
        for d in range(N_DEV):
            src = lax.rem(my - d + N_DEV, N_DEV)
            slot = d % 2
            if d + 1 < N_DEV:
                for cp in w_copies(d + 1, (d + 1) % 2):
                    cp.start()
            with _scope(f"wait_w#d={d}"):
                for cp in w_copies(d, slot):
                    cp.wait()
            with _scope(f"wait_x#d={d}"):
                if d > 0 and not _SKIP_A2A:
                    x_recv_rdma(d).wait_recv()

            with _scope(f"dots#d={d}"):
                x_chunk = x_rows[:, pl.ds(src * KC, KC)].astype(jnp.float32)
                for c in range(0, N, NB):
                    part = jnp.dot(
                        x_chunk, w_bufs[slot, :, pl.ds(c, NB)],
                        preferred_element_type=jnp.float32,
                    )
                    if d == 0:
                        out_ref[:, pl.ds(c, NB)] = part
                    elif d < N_DEV - 1:
                        out_ref[:, pl.ds(c, NB)] = out_ref[:, pl.ds(c, NB)] + part
                    else:
                        v = jnp.maximum(out_ref[:, pl.ds(c, NB)] + part, 0.0)
                        out_ref[:, pl.ds(c, NB)] = v
                        mloc = jnp.maximum(mloc, jnp.max(v))

        if not _SKIP_A2A:
            for d in range(1, N_DEV):
                x_send_rdma(d).wait_send()

        gather[pl.ds(0, 1), :] = jnp.full((1, 128), mloc, jnp.float32)
        acc[0] = mloc
        if not _SKIP_AMAX:
            with _scope("amax_exchange"):
                for d in range(1, N_DEV):
                    amax_send_rdma(d).start()
                for d in range(1, N_DEV):
                    amax_send_rdma(d).wait()

        g = jnp.maximum(jnp.max(gather[:, :]), 1e-30)
        scale = g / 448.0
        if not _SKIP_QUANT:
            with _scope("quant"):
                for c in range(0, N, 2048):
                    blk = out_ref[:, pl.ds(c, 2048)]
                    q = (blk * (448.0 / g)).astype(jnp.float8_e4m3fn)
                    out_ref[:, pl.ds(c, 2048)] = q.astype(jnp.float32) * scale

    return pl.pallas_call(
        body,
        out_shape=jax.ShapeDtypeStruct((M_PER, N), jnp.float32),
        in_specs=[
            pl.BlockSpec(memory_space=pl.ANY),
            pl.BlockSpec(memory_space=pl.ANY),
        ],
        out_specs=pl.BlockSpec(memory_space=pltpu.VMEM),
        scratch_shapes=[
            pltpu.VMEM((M_PER, K), jnp.bfloat16),
            pltpu.VMEM((2, KC, N), jnp.float32),
            pltpu.VMEM((K, KC), jnp.bfloat16),
            pltpu.VMEM((2, M_PER, KC), jnp.float32),
            pltpu.VMEM((N_DEV, 128), jnp.float32),
            pltpu.SMEM((1,), jnp.float32),
            pltpu.SemaphoreType.DMA((N_DEV,)),
            pltpu.SemaphoreType.DMA((N_DEV,)),
            pltpu.SemaphoreType.DMA((N_DEV,)),
            pltpu.SemaphoreType.DMA((N_DEV,)),
            pltpu.SemaphoreType.DMA((2, 4)),
            pltpu.SemaphoreType.DMA((2,)),
        ],
        compiler_params=pltpu.CompilerParams(
            collective_id=0,
            vmem_limit_bytes=63 * 1024 * 1024,
        ),
    )(x, w_mat)
